# baseline (speedup 1.0000x reference)
import jax
import jax.numpy as jnp
from jax.experimental import pallas as pl
from jax.experimental.pallas import tpu as pltpu

_N = 20000
_D = 1024
_C = 320
_CT = _C + 1
_CHUNK = 1000
_NBUF = 4
_NBLK = _N // _CHUNK


def _fused_heads_kernel(x_hbm, wc_ref, bc_ref, od_ref, oi_ref, xbuf, insem):
    i = pl.program_id(0)

    def start_in(chunk, slot):
        pltpu.make_async_copy(
            x_hbm.at[pl.ds(chunk * _CHUNK, _CHUNK), :],
            xbuf.at[slot],
            insem.at[slot],
        ).start()

    @pl.when(i == 0)
    def _warmup():
        for c in range(_NBUF):
            start_in(c, c)

    slot = jax.lax.rem(i, _NBUF)
    pltpu.make_async_copy(
        x_hbm.at[pl.ds(i * _CHUNK, _CHUNK), :], xbuf.at[slot], insem.at[slot]
    ).wait()

    xb = xbuf[slot].astype(jnp.bfloat16)
    acc = (
        jnp.dot(xb, wc_ref[...], preferred_element_type=jnp.float32)
        + bc_ref[...]
    )
    od_ref[...] = acc[:, :_C]
    oi_ref[...] = acc[:, _C:_CT]

    @pl.when(i + _NBUF < _NBLK)
    def _prefetch():
        start_in(i + _NBUF, slot)


def kernel(x, W_bbox, b_bbox, W_iou, b_iou):
    if x.ndim > 2:
        x = x.reshape(x.shape[0], -1)
    wc = jnp.concatenate([W_bbox, W_iou], axis=1).astype(jnp.bfloat16)
    bc = jnp.concatenate([b_bbox, b_iou]).reshape(1, _CT)

    out_shapes = (
        jax.ShapeDtypeStruct((_N, _C), jnp.float32),
        jax.ShapeDtypeStruct((_N, 1), jnp.float32),
    )
    od, oi = pl.pallas_call(
        _fused_heads_kernel,
        grid=(_NBLK,),
        in_specs=[
            pl.BlockSpec(memory_space=pltpu.MemorySpace.HBM),
            pl.BlockSpec((_D, _CT), lambda i: (0, 0)),
            pl.BlockSpec((1, _CT), lambda i: (0, 0)),
        ],
        out_specs=(
            pl.BlockSpec((_CHUNK, _C), lambda i: (i, 0)),
            pl.BlockSpec((_CHUNK, 1), lambda i: (i, 0)),
        ),
        out_shape=out_shapes,
        scratch_shapes=[
            pltpu.VMEM((_NBUF, _CHUNK, _D), jnp.float32),
            pltpu.SemaphoreType.DMA((_NBUF,)),
        ],
        compiler_params=pltpu.CompilerParams(
            dimension_semantics=("arbitrary",),
        ),
    )(x, wc, bc)
    return (od, oi)
